# Initial kernel scaffold; baseline (speedup 1.0000x reference)
import jax
import jax.numpy as jnp
from jax.experimental import pallas as pl


def kernel(x, edge_index, W1, b1, W2, b2, W_out, b_out):
    raise NotImplementedError("write your pallas kernel here")


# speedup vs baseline: 14.8844x; 14.8844x over previous
import functools

import jax
import jax.numpy as jnp
from jax import lax
from jax.experimental import pallas as pl
from jax.experimental.pallas import tpu as pltpu
from jax.experimental.pallas import tpu_sc as plsc

_N = 10000
_D = 128
_E = 320000
_NC = 2
_NS = 16
_NW = _NC * _NS
_EPW = _E // _NW
_CH = 128
_NFULL = _EPW // _CH
_TAIL = _EPW - _NFULL * _CH
_NPAD = 10240
_RPS = _NPAD // _NS

_mesh = plsc.VectorSubcoreMesh(core_axis_name="c", subcore_axis_name="s")


@functools.partial(
    pl.kernel,
    mesh=_mesh,
    out_type=jax.ShapeDtypeStruct((_NC, _NPAD), jnp.float32),
    scratch_types=[
        pltpu.VMEM((_CH,), jnp.int32),
        pltpu.VMEM((_TAIL,), jnp.int32),
        pltpu.VMEM((_CH,), jnp.float32),
        pltpu.VMEM((_TAIL,), jnp.float32),
        pltpu.VMEM_SHARED((_NPAD,), jnp.float32),
    ],
)
def _deg_kernel(dst_hbm, ones_hbm, zvec_hbm, out_hbm,
                dst_v, dst_t, ones_v, ones_t, acc):
    c = lax.axis_index("c")
    s = lax.axis_index("s")
    w = c * _NS + s
    pltpu.sync_copy(ones_hbm, ones_v)
    pltpu.sync_copy(ones_hbm.at[pl.ds(0, _TAIL)], ones_t)
    pltpu.sync_copy(zvec_hbm, acc.at[pl.ds(s * _RPS, _RPS)])
    plsc.subcore_barrier()
    base = w * _EPW

    def body(k, carry):
        pltpu.sync_copy(dst_hbm.at[pl.ds(base + k * _CH, _CH)], dst_v)
        pltpu.sync_copy(ones_v, acc.at[dst_v], add=True)
        return carry

    lax.fori_loop(0, _NFULL, body, 0)
    pltpu.sync_copy(dst_hbm.at[pl.ds(base + _NFULL * _CH, _TAIL)], dst_t)
    pltpu.sync_copy(ones_t, acc.at[dst_t], add=True)
    plsc.subcore_barrier()
    pltpu.sync_copy(acc.at[pl.ds(s * _RPS, _RPS)],
                    out_hbm.at[c, pl.ds(s * _RPS, _RPS)])


@functools.partial(
    pl.kernel,
    mesh=_mesh,
    out_type=jax.ShapeDtypeStruct((_NC, _NPAD, _D), jnp.float32),
    scratch_types=[
        pltpu.VMEM((_CH,), jnp.int32),
        pltpu.VMEM((_CH,), jnp.int32),
        pltpu.VMEM((_CH, _D), jnp.float32),
        pltpu.VMEM((_TAIL,), jnp.int32),
        pltpu.VMEM((_TAIL,), jnp.int32),
        pltpu.VMEM((_TAIL, _D), jnp.float32),
        pltpu.VMEM_SHARED((_NPAD, _D), jnp.float32),
        pltpu.SemaphoreType.DMA,
    ],
)
def _agg_kernel(hp_hbm, src_hbm, dst_hbm, zrow_hbm, out_hbm,
                src_v, dst_v, rows_v, src_t, dst_t, rows_t, acc, sem):
    c = lax.axis_index("c")
    s = lax.axis_index("s")
    w = c * _NS + s
    pltpu.sync_copy(zrow_hbm, acc.at[pl.ds(s * _RPS, _RPS)])
    plsc.subcore_barrier()
    base = w * _EPW

    def body(k, carry):
        e0 = base + k * _CH
        pltpu.sync_copy(src_hbm.at[pl.ds(e0, _CH)], src_v)
        pltpu.sync_copy(dst_hbm.at[pl.ds(e0, _CH)], dst_v)
        pltpu.async_copy(hp_hbm.at[src_v], rows_v, sem).wait()
        pltpu.sync_copy(rows_v, acc.at[dst_v], add=True)
        return carry

    lax.fori_loop(0, _NFULL, body, 0)
    e0 = base + _NFULL * _CH
    pltpu.sync_copy(src_hbm.at[pl.ds(e0, _TAIL)], src_t)
    pltpu.sync_copy(dst_hbm.at[pl.ds(e0, _TAIL)], dst_t)
    pltpu.async_copy(hp_hbm.at[src_t], rows_t, sem).wait()
    pltpu.sync_copy(rows_t, acc.at[dst_t], add=True)
    plsc.subcore_barrier()
    pltpu.sync_copy(acc.at[pl.ds(s * _RPS, _RPS)],
                    out_hbm.at[c, pl.ds(s * _RPS, _RPS)])


def _tc1_body(x_ref, w_ref, degp_ref, hp_ref, dinv_ref):
    deg = degp_ref[0] + degp_ref[1] + 1.0
    dinv = lax.rsqrt(deg)
    dinv_ref[...] = dinv
    h = jnp.dot(x_ref[...], w_ref[...], preferred_element_type=jnp.float32)
    hp_ref[...] = h * dinv


_tc1 = pl.pallas_call(
    _tc1_body,
    out_shape=(jax.ShapeDtypeStruct((_N, _D), jnp.float32),
               jax.ShapeDtypeStruct((_N, 1), jnp.float32)),
)


def _tc2_body(aggp_ref, hp_ref, dinv_ref, b_ref, res_ref, w_ref,
              h_ref, hpn_ref):
    dinv = dinv_ref[...]
    agg = aggp_ref[0][:_N] + aggp_ref[1][:_N]
    z = dinv * (agg + hp_ref[...]) + b_ref[...]
    h = jnp.maximum(z, 0.0) + res_ref[...]
    h_ref[...] = h
    hpn_ref[...] = jnp.dot(h, w_ref[...],
                           preferred_element_type=jnp.float32) * dinv


_tc2 = pl.pallas_call(
    _tc2_body,
    out_shape=(jax.ShapeDtypeStruct((_N, _D), jnp.float32),
               jax.ShapeDtypeStruct((_N, _D), jnp.float32)),
)


def _tc3_body(aggp_ref, hp_ref, dinv_ref, b_ref, res_ref, wout_ref,
              bout_ref, out_ref):
    dinv = dinv_ref[...]
    agg = aggp_ref[0][:_N] + aggp_ref[1][:_N]
    z = dinv * (agg + hp_ref[...]) + b_ref[...]
    h = jnp.maximum(z, 0.0) + res_ref[...]
    out_ref[...] = jnp.dot(h, wout_ref[...],
                           preferred_element_type=jnp.float32) + bout_ref[...]


_tc3 = pl.pallas_call(
    _tc3_body,
    out_shape=jax.ShapeDtypeStruct((_N, _D), jnp.float32),
)


def kernel(x, edge_index, W1, b1, W2, b2, W_out, b_out):
    src = edge_index[0]
    dst = edge_index[1]
    ones = jnp.ones((_CH,), jnp.float32)
    zvec = jnp.zeros((_RPS,), jnp.float32)
    zrow = jnp.zeros((_RPS, _D), jnp.float32)

    degp = _deg_kernel(dst, ones, zvec)
    degp3 = degp[:, :_N][:, :, None]

    hp1, dinv = _tc1(x, W1, degp3)
    agg1 = _agg_kernel(hp1, src, dst, zrow)
    h1, hp2 = _tc2(agg1, hp1, dinv, b1.reshape(1, _D), x, W2)
    agg2 = _agg_kernel(hp2, src, dst, zrow)
    out = _tc3(agg2, hp2, dinv, b2.reshape(1, _D), h1, W_out,
               b_out.reshape(1, _D))
    return out


# speedup vs baseline: 27.3978x; 1.8407x over previous
import functools

import jax
import jax.numpy as jnp
from jax import lax
from jax.experimental import pallas as pl
from jax.experimental.pallas import tpu as pltpu
from jax.experimental.pallas import tpu_sc as plsc

_N = 10000
_D = 128
_E = 320000
_NC = 2
_NS = 16
_NW = _NC * _NS
_CH = 80
_CPT = _E // _NW // _CH
_NPAD = 10240
_RPS = _NPAD // _NS

_mesh = plsc.VectorSubcoreMesh(core_axis_name="c", subcore_axis_name="s")


@functools.partial(
    pl.kernel,
    mesh=_mesh,
    out_type=jax.ShapeDtypeStruct((_NC, _NPAD), jnp.float32),
    scratch_types=[
        pltpu.VMEM((_CPT, _CH), jnp.int32),
        pltpu.VMEM((_CH,), jnp.float32),
        pltpu.VMEM_SHARED((_NPAD,), jnp.float32),
    ],
)
def _deg_kernel(dsti_hbm, ones_hbm, zvec_hbm, out_hbm, dsti_v, ones_v, acc):
    c = lax.axis_index("c")
    s = lax.axis_index("s")
    w = c * _NS + s
    pltpu.sync_copy(ones_hbm, ones_v)
    pltpu.sync_copy(zvec_hbm, acc.at[pl.ds(s * _RPS, _RPS)])
    pltpu.sync_copy(dsti_hbm.at[w], dsti_v)
    plsc.subcore_barrier()

    def body(k, carry):
        pltpu.sync_copy(ones_v, acc.at[dsti_v.at[k]], add=True)
        return carry

    lax.fori_loop(0, _CPT, body, 0)
    plsc.subcore_barrier()
    pltpu.sync_copy(acc.at[pl.ds(s * _RPS, _RPS)],
                    out_hbm.at[c, pl.ds(s * _RPS, _RPS)])


@functools.partial(
    pl.kernel,
    mesh=_mesh,
    out_type=jax.ShapeDtypeStruct((_NC, _NPAD, _D), jnp.float32),
    scratch_types=[
        pltpu.VMEM((_CPT * _CH,), jnp.int32),
        pltpu.VMEM((_CH,), jnp.int32),
        pltpu.VMEM((_CH,), jnp.int32),
        pltpu.VMEM((_CH, _D), jnp.float32),
        pltpu.VMEM((_CH, _D), jnp.float32),
        pltpu.VMEM_SHARED((_NPAD, _D), jnp.float32),
        pltpu.SemaphoreType.DMA,
        pltpu.SemaphoreType.DMA,
        pltpu.SemaphoreType.DMA,
        pltpu.SemaphoreType.DMA,
    ],
)
def _agg_kernel(hp_hbm, srci_hbm, dsti_hbm, zrow_hbm, out_hbm,
                srci_v, dst0, dst1, rows0, rows1, acc, g0, g1, d0, d1):
    c = lax.axis_index("c")
    s = lax.axis_index("s")
    w = c * _NS + s
    pltpu.sync_copy(zrow_hbm, acc.at[pl.ds(s * _RPS, _RPS)])
    ebase = w * (_CPT * _CH)
    pltpu.sync_copy(srci_hbm.at[pl.ds(ebase, _CPT * _CH)], srci_v)

    def dstload(k, buf, sem):
        pltpu.async_copy(dsti_hbm.at[pl.ds(ebase + k * _CH, _CH)], buf, sem)

    def dst_wait(buf, sem):
        pltpu.make_async_copy(dsti_hbm.at[pl.ds(0, _CH)], buf, sem).wait()

    def gather(k, buf, sem):
        pltpu.async_copy(hp_hbm.at[srci_v.at[pl.ds(k * _CH, _CH)]], buf, sem)

    def gather_wait(buf, sem):
        pltpu.make_async_copy(hp_hbm.at[srci_v.at[pl.ds(0, _CH)]],
                              buf, sem).wait()

    dstload(0, dst0, d0)
    dstload(1, dst1, d1)
    plsc.subcore_barrier()

    def scat(buf, dbuf):
        pltpu.sync_copy(buf, acc.at[dbuf], add=True)

    gather(0, rows0, g0)

    def body(i, carry):
        c0 = 2 * i
        gather(c0 + 1, rows1, g1)
        dst_wait(dst0, d0)
        gather_wait(rows0, g0)
        scat(rows0, dst0)
        dstload(c0 + 2, dst0, d0)
        gather(c0 + 2, rows0, g0)
        dst_wait(dst1, d1)
        gather_wait(rows1, g1)
        scat(rows1, dst1)
        dstload(c0 + 3, dst1, d1)
        return carry

    lax.fori_loop(0, (_CPT - 3) // 2, body, 0)
    gather(_CPT - 2, rows1, g1)
    dst_wait(dst0, d0)
    gather_wait(rows0, g0)
    scat(rows0, dst0)
    dstload(_CPT - 1, dst0, d0)
    gather(_CPT - 1, rows0, g0)
    dst_wait(dst1, d1)
    gather_wait(rows1, g1)
    scat(rows1, dst1)
    dst_wait(dst0, d0)
    gather_wait(rows0, g0)
    scat(rows0, dst0)
    plsc.subcore_barrier()
    pltpu.sync_copy(acc.at[pl.ds(s * _RPS, _RPS)],
                    out_hbm.at[c, pl.ds(s * _RPS, _RPS)])


def _tc1_body(x_ref, w_ref, degp_ref, hp_ref, dinv_ref):
    deg = degp_ref[0] + degp_ref[1] + 1.0
    dinv = lax.rsqrt(deg)
    dinv_ref[...] = dinv
    h = jnp.dot(x_ref[...], w_ref[...], preferred_element_type=jnp.float32)
    hp_ref[...] = h * dinv


_tc1 = pl.pallas_call(
    _tc1_body,
    out_shape=(jax.ShapeDtypeStruct((_N, _D), jnp.float32),
               jax.ShapeDtypeStruct((_N, 1), jnp.float32)),
)


def _tc2_body(aggp_ref, hp_ref, dinv_ref, b_ref, res_ref, w_ref,
              h_ref, hpn_ref):
    dinv = dinv_ref[...]
    agg = aggp_ref[0][:_N] + aggp_ref[1][:_N]
    z = dinv * (agg + hp_ref[...]) + b_ref[...]
    h = jnp.maximum(z, 0.0) + res_ref[...]
    h_ref[...] = h
    hpn_ref[...] = jnp.dot(h, w_ref[...],
                           preferred_element_type=jnp.float32) * dinv


_tc2 = pl.pallas_call(
    _tc2_body,
    out_shape=(jax.ShapeDtypeStruct((_N, _D), jnp.float32),
               jax.ShapeDtypeStruct((_N, _D), jnp.float32)),
)


def _tc3_body(aggp_ref, hp_ref, dinv_ref, b_ref, res_ref, wout_ref,
              bout_ref, out_ref):
    dinv = dinv_ref[...]
    agg = aggp_ref[0][:_N] + aggp_ref[1][:_N]
    z = dinv * (agg + hp_ref[...]) + b_ref[...]
    h = jnp.maximum(z, 0.0) + res_ref[...]
    out_ref[...] = jnp.dot(h, wout_ref[...],
                           preferred_element_type=jnp.float32) + bout_ref[...]


_tc3 = pl.pallas_call(
    _tc3_body,
    out_shape=jax.ShapeDtypeStruct((_N, _D), jnp.float32),
)


def kernel(x, edge_index, W1, b1, W2, b2, W_out, b_out):
    src = edge_index[0]
    dst = edge_index[1]
    dst3d = dst.reshape(_NW, _CPT, _CH)
    ones = jnp.ones((_CH,), jnp.float32)
    zvec = jnp.zeros((_RPS,), jnp.float32)
    zrow = jnp.zeros((_RPS, _D), jnp.float32)

    degp = _deg_kernel(dst3d, ones, zvec)
    degp3 = degp[:, :_N][:, :, None]

    hp1, dinv = _tc1(x, W1, degp3)
    agg1 = _agg_kernel(hp1, src, dst, zrow)
    h1, hp2 = _tc2(agg1, hp1, dinv, b1.reshape(1, _D), x, W2)
    agg2 = _agg_kernel(hp2, src, dst, zrow)
    out = _tc3(agg2, hp2, dinv, b2.reshape(1, _D), h1, W_out,
               b_out.reshape(1, _D))
    return out


# speedup vs baseline: 28.2097x; 1.0296x over previous
import functools

import jax
import jax.numpy as jnp
from jax import lax
from jax.experimental import pallas as pl
from jax.experimental.pallas import tpu as pltpu
from jax.experimental.pallas import tpu_sc as plsc

_N = 10000
_D = 128
_E = 320000
_NC = 2
_NS = 16
_NW = _NC * _NS
_CH = 80
_CPT = _E // _NW // _CH
_NPAD = 10240
_RPS = _NPAD // _NS

_mesh = plsc.VectorSubcoreMesh(core_axis_name="c", subcore_axis_name="s")


@functools.partial(
    pl.kernel,
    mesh=_mesh,
    out_type=jax.ShapeDtypeStruct((_NC, _NPAD), jnp.float32),
    scratch_types=[
        pltpu.VMEM((_CPT, _CH), jnp.int32),
        pltpu.VMEM((_CH,), jnp.float32),
        pltpu.VMEM_SHARED((_NPAD,), jnp.float32),
    ],
)
def _deg_kernel(dsti_hbm, ones_hbm, zvec_hbm, out_hbm, dsti_v, ones_v, acc):
    c = lax.axis_index("c")
    s = lax.axis_index("s")
    w = c * _NS + s
    pltpu.sync_copy(ones_hbm, ones_v)
    pltpu.sync_copy(zvec_hbm, acc.at[pl.ds(s * _RPS, _RPS)])
    pltpu.sync_copy(dsti_hbm.at[w], dsti_v)
    plsc.subcore_barrier()

    def body(k, carry):
        pltpu.sync_copy(ones_v, acc.at[dsti_v.at[k]], add=True)
        return carry

    lax.fori_loop(0, _CPT, body, 0)
    plsc.subcore_barrier()
    pltpu.sync_copy(acc.at[pl.ds(s * _RPS, _RPS)],
                    out_hbm.at[c, pl.ds(s * _RPS, _RPS)])


@functools.partial(
    pl.kernel,
    mesh=_mesh,
    out_type=jax.ShapeDtypeStruct((_NC, _NPAD, _D), jnp.float32),
    scratch_types=(
        [pltpu.VMEM((_CH,), jnp.int32)] * 4
        + [pltpu.VMEM((_CH,), jnp.int32)] * 4
        + [pltpu.VMEM((_CH, _D), jnp.float32)] * 4
        + [pltpu.VMEM_SHARED((_NPAD, _D), jnp.float32)]
        + [pltpu.SemaphoreType.DMA] * 16
    ),
)
def _agg_kernel(hp_hbm, srci_hbm, dsti_hbm, zrow_hbm, out_hbm, *refs):
    srcb = refs[0:4]
    dstb = refs[4:8]
    rows = refs[8:12]
    acc = refs[12]
    si = refs[13:17]
    di = refs[17:21]
    gs = refs[21:25]
    ss = refs[25:29]
    c = lax.axis_index("c")
    s = lax.axis_index("s")
    w = c * _NS + s
    pltpu.sync_copy(zrow_hbm, acc.at[pl.ds(s * _RPS, _RPS)])
    ebase = w * (_CPT * _CH)

    def idxload(hbm, k, buf, sem):
        pltpu.async_copy(hbm.at[pl.ds(ebase + k * _CH, _CH)], buf, sem)

    def idx_wait(hbm, buf, sem):
        pltpu.make_async_copy(hbm.at[pl.ds(0, _CH)], buf, sem).wait()

    def gather(b):
        pltpu.async_copy(hp_hbm.at[srcb[b]], rows[b], gs[b])

    def gather_wait(b):
        pltpu.make_async_copy(hp_hbm.at[srcb[b]], rows[b], gs[b]).wait()

    def scat_wait(b):
        pltpu.make_async_copy(rows[b], acc.at[dstb[b]], ss[b]).wait()

    for b in range(4):
        idxload(srci_hbm, b, srcb[b], si[b])
        idxload(dsti_hbm, b, dstb[b], di[b])
    plsc.subcore_barrier()
    for b in range(4):
        idx_wait(srci_hbm, srcb[b], si[b])
        gather(b)

    def body(i, carry):
        for b in range(4):
            kn = jnp.minimum(4 * i + 4 + b, _CPT - 1)
            idx_wait(dsti_hbm, dstb[b], di[b])
            gather_wait(b)
            pltpu.async_copy(rows[b], acc.at[dstb[b]], ss[b], add=True)
            idxload(srci_hbm, kn, srcb[b], si[b])
        for b in range(4):
            kn = jnp.minimum(4 * i + 4 + b, _CPT - 1)
            scat_wait(b)
            idxload(dsti_hbm, kn, dstb[b], di[b])
            idx_wait(srci_hbm, srcb[b], si[b])
            gather(b)
        return carry

    lax.fori_loop(0, (_CPT - 1) // 4, body, 0)
    idx_wait(dsti_hbm, dstb[0], di[0])
    gather_wait(0)
    pltpu.sync_copy(rows[0], acc.at[dstb[0]], add=True)
    for b in range(1, 4):
        idx_wait(dsti_hbm, dstb[b], di[b])
        gather_wait(b)
    plsc.subcore_barrier()
    pltpu.sync_copy(acc.at[pl.ds(s * _RPS, _RPS)],
                    out_hbm.at[c, pl.ds(s * _RPS, _RPS)])


def _tc1_body(x_ref, w_ref, degp_ref, hp_ref, dinv_ref):
    deg = degp_ref[0] + degp_ref[1] + 1.0
    dinv = lax.rsqrt(deg)
    dinv_ref[...] = dinv
    h = jnp.dot(x_ref[...], w_ref[...], preferred_element_type=jnp.float32)
    hp_ref[...] = h * dinv


_tc1 = pl.pallas_call(
    _tc1_body,
    out_shape=(jax.ShapeDtypeStruct((_N, _D), jnp.float32),
               jax.ShapeDtypeStruct((_N, 1), jnp.float32)),
)


def _tc2_body(aggp_ref, hp_ref, dinv_ref, b_ref, res_ref, w_ref,
              h_ref, hpn_ref):
    dinv = dinv_ref[...]
    agg = aggp_ref[0][:_N] + aggp_ref[1][:_N]
    z = dinv * (agg + hp_ref[...]) + b_ref[...]
    h = jnp.maximum(z, 0.0) + res_ref[...]
    h_ref[...] = h
    hpn_ref[...] = jnp.dot(h, w_ref[...],
                           preferred_element_type=jnp.float32) * dinv


_tc2 = pl.pallas_call(
    _tc2_body,
    out_shape=(jax.ShapeDtypeStruct((_N, _D), jnp.float32),
               jax.ShapeDtypeStruct((_N, _D), jnp.float32)),
)


def _tc3_body(aggp_ref, hp_ref, dinv_ref, b_ref, res_ref, wout_ref,
              bout_ref, out_ref):
    dinv = dinv_ref[...]
    agg = aggp_ref[0][:_N] + aggp_ref[1][:_N]
    z = dinv * (agg + hp_ref[...]) + b_ref[...]
    h = jnp.maximum(z, 0.0) + res_ref[...]
    out_ref[...] = jnp.dot(h, wout_ref[...],
                           preferred_element_type=jnp.float32) + bout_ref[...]


_tc3 = pl.pallas_call(
    _tc3_body,
    out_shape=jax.ShapeDtypeStruct((_N, _D), jnp.float32),
)


def kernel(x, edge_index, W1, b1, W2, b2, W_out, b_out):
    src = edge_index[0]
    dst = edge_index[1]
    dst3d = dst.reshape(_NW, _CPT, _CH)
    ones = jnp.ones((_CH,), jnp.float32)
    zvec = jnp.zeros((_RPS,), jnp.float32)
    zrow = jnp.zeros((_RPS, _D), jnp.float32)

    degp = _deg_kernel(dst3d, ones, zvec)
    degp3 = degp[:, :_N][:, :, None]

    hp1, dinv = _tc1(x, W1, degp3)
    agg1 = _agg_kernel(hp1, src, dst, zrow)
    h1, hp2 = _tc2(agg1, hp1, dinv, b1.reshape(1, _D), x, W2)
    agg2 = _agg_kernel(hp2, src, dst, zrow)
    out = _tc3(agg2, hp2, dinv, b2.reshape(1, _D), h1, W_out,
               b_out.reshape(1, _D))
    return out
